# baseline (speedup 1.0000x reference)
import jax
import jax.numpy as jnp
from jax.experimental import pallas as pl

_BLK = 1000


def _cell_body(x_ref, h_ref, c_ref, w4_ref, r4_ref, bias4_ref,
               wci_ref, wcf_ref, wco_ref, lin_w_ref, lin_b_ref,
               out_ref, h_out_ref, c_out_ref):
    x = x_ref[...]
    h = h_ref[...]
    c = c_ref[...]
    g = jnp.dot(x, w4_ref[...], preferred_element_type=jnp.float32)
    g = g + jnp.dot(h, r4_ref[...], preferred_element_type=jnp.float32)
    g = g + bias4_ref[...]
    gate_i = jax.nn.sigmoid(g[:, 0:32] + wci_ref[...] * c)
    gate_f = jax.nn.sigmoid(g[:, 32:64] + wcf_ref[...] * c)
    gate_t = jnp.tanh(g[:, 64:96])
    c_new = gate_f * c + gate_i * gate_t
    gate_o = jax.nn.sigmoid(g[:, 96:128] + wco_ref[...] * c_new)
    h_new = gate_o * jnp.tanh(c_new)
    out_ref[...] = (jnp.dot(h_new, lin_w_ref[...],
                            preferred_element_type=jnp.float32)
                    + lin_b_ref[...])
    h_out_ref[...] = h_new
    c_out_ref[...] = c_new


def kernel(x, edge_index, edge_weight, h, c, W_i, W_f, W_c, W_o,
           conv_i_w, conv_i_b, conv_f_w, conv_f_b,
           conv_c_w, conv_c_b, conv_o_w, conv_o_b,
           w_c_i, w_c_f, w_c_o,
           b_i, b_f, b_c, b_o,
           lin_w, lin_b):
    del edge_index, edge_weight
    n, f_in = x.shape
    f_out = h.shape[1]

    w4 = jnp.concatenate([W_i, W_f, W_c, W_o], axis=1)
    r4 = jnp.concatenate([conv_i_w, conv_f_w, conv_c_w, conv_o_w], axis=1)
    bias4 = jnp.concatenate(
        [conv_i_b[None, :] + b_i, conv_f_b[None, :] + b_f,
         conv_c_b[None, :] + b_c, conv_o_b[None, :] + b_o], axis=1)
    lin_b2 = lin_b.reshape(1, 1)

    grid = (n // _BLK,)
    row_blk = lambda i: (i, 0)
    bcast = lambda i: (0, 0)

    out, h_new, c_new = pl.pallas_call(
        _cell_body,
        grid=grid,
        in_specs=[
            pl.BlockSpec((_BLK, f_in), row_blk),
            pl.BlockSpec((_BLK, f_out), row_blk),
            pl.BlockSpec((_BLK, f_out), row_blk),
            pl.BlockSpec((f_in, 4 * f_out), bcast),
            pl.BlockSpec((f_out, 4 * f_out), bcast),
            pl.BlockSpec((1, 4 * f_out), bcast),
            pl.BlockSpec((1, f_out), bcast),
            pl.BlockSpec((1, f_out), bcast),
            pl.BlockSpec((1, f_out), bcast),
            pl.BlockSpec((f_out, 1), bcast),
            pl.BlockSpec((1, 1), bcast),
        ],
        out_specs=[
            pl.BlockSpec((_BLK, 1), row_blk),
            pl.BlockSpec((_BLK, f_out), row_blk),
            pl.BlockSpec((_BLK, f_out), row_blk),
        ],
        out_shape=[
            jax.ShapeDtypeStruct((n, 1), jnp.float32),
            jax.ShapeDtypeStruct((n, f_out), jnp.float32),
            jax.ShapeDtypeStruct((n, f_out), jnp.float32),
        ],
    )(x, h, c, w4, r4, bias4, w_c_i, w_c_f, w_c_o, lin_w, lin_b2)

    return (out, h_new, c_new)
